# Initial kernel scaffold; baseline (speedup 1.0000x reference)
import functools

import jax
import jax.numpy as jnp
from jax import lax
from jax.experimental import pallas as pl
from jax.experimental.pallas import tpu as pltpu
from jax.experimental.pallas import tpu_sc as plsc

NC, NS, L = 2, 16, 16
NW = NC * NS
CHUNK = 125
DEGW = 16


def _sc_aggregate(x, src3, dst3, n_nodes, d_feat, n_chunks):
    rows_per_tile = n_nodes // NS
    mesh = plsc.VectorSubcoreMesh(core_axis_name="c", subcore_axis_name="s")

    @functools.partial(
        pl.kernel,
        mesh=mesh,
        out_type=[
            jax.ShapeDtypeStruct((NC, n_nodes, d_feat), jnp.float32),
            jax.ShapeDtypeStruct((NC, n_nodes, DEGW), jnp.float32),
        ],
        scratch_types=[
            pltpu.VMEM((n_chunks, CHUNK), jnp.int32),
            pltpu.VMEM((n_chunks, CHUNK), jnp.int32),
            pltpu.VMEM((CHUNK, d_feat), jnp.float32),
            pltpu.VMEM((CHUNK, DEGW), jnp.float32),
            pltpu.VMEM((CHUNK, d_feat), jnp.float32),
            pltpu.VMEM((n_nodes // NS, DEGW), jnp.float32),
            pltpu.VMEM_SHARED((n_nodes, d_feat), jnp.float32),
            pltpu.VMEM_SHARED((n_nodes, DEGW), jnp.float32),
        ],
    )
    def agg_kernel(x_hbm, src_hbm, dst_hbm, acc_out, deg_out,
                   src_v, dst_v, rows_v, ones_v, zrow_v, zdeg_v,
                   acc_s, deg_s):
        cid = lax.axis_index("c")
        sid = lax.axis_index("s")
        wid = sid * NC + cid
        base = sid * rows_per_tile

        zero16 = jnp.zeros((L,), jnp.float32)
        one16 = jnp.ones((L,), jnp.float32)

        def fill_zrow(i, _):
            r = i // (d_feat // L)
            c = i % (d_feat // L)
            zrow_v[r, pl.ds(c * L, L)] = zero16
            return 0
        lax.fori_loop(0, CHUNK * (d_feat // L), fill_zrow, 0)

        def fill_small(r, _):
            ones_v[r, :] = one16
            return 0
        lax.fori_loop(0, CHUNK, fill_small, 0)

        def fill_zdeg(r, _):
            zdeg_v[r, :] = zero16
            return 0
        lax.fori_loop(0, rows_per_tile, fill_zdeg, 0)

        n_ztiles = rows_per_tile // CHUNK
        for k in range(n_ztiles):
            pltpu.sync_copy(zrow_v, acc_s.at[pl.ds(base + k * CHUNK, CHUNK)])
        rem = rows_per_tile - n_ztiles * CHUNK
        if rem:
            pltpu.sync_copy(zrow_v.at[pl.ds(0, rem)],
                            acc_s.at[pl.ds(base + n_ztiles * CHUNK, rem)])
        pltpu.sync_copy(zdeg_v, deg_s.at[pl.ds(base, rows_per_tile)])
        plsc.subcore_barrier()

        pltpu.sync_copy(src_hbm.at[wid], src_v)
        pltpu.sync_copy(dst_hbm.at[wid], dst_v)

        def chunk_body(j, _):
            pltpu.sync_copy(x_hbm.at[src_v.at[j]], rows_v)
            pltpu.sync_copy(rows_v, acc_s.at[dst_v.at[j]], add=True)
            pltpu.sync_copy(ones_v, deg_s.at[dst_v.at[j]], add=True)
            return 0
        lax.fori_loop(0, n_chunks, chunk_body, 0)
        plsc.subcore_barrier()

        pltpu.sync_copy(acc_s.at[pl.ds(base, rows_per_tile)],
                        acc_out.at[cid, pl.ds(base, rows_per_tile)])
        pltpu.sync_copy(deg_s.at[pl.ds(base, rows_per_tile)],
                        deg_out.at[cid, pl.ds(base, rows_per_tile)])

    return agg_kernel(x, src3, dst3)


def _tc_fused(acc2, deg2, x, batch3, wcat, bl2, wlin_t, blin2,
              n_nodes, d_feat, h_feat, f_out, n_graphs, blk):
    nsteps = n_nodes // blk
    two_h = 2 * h_feat

    def body(acc_ref, deg_ref, x_ref, batch_ref, wcat_ref, bl_ref,
             wlin_ref, blin_ref, out_ref, max_s, sum_s, cnt_s):
        i = pl.program_id(0)

        @pl.when(i == 0)
        def _init():
            max_s[...] = jnp.full((n_graphs, h_feat), -jnp.inf, jnp.float32)
            sum_s[...] = jnp.zeros((n_graphs, h_feat), jnp.float32)
            cnt_s[...] = jnp.zeros((n_graphs, h_feat), jnp.float32)

        a = acc_ref[0] + acc_ref[1]
        d = deg_ref[0, :, 0] + deg_ref[1, :, 0]
        agg = a / jnp.maximum(d, 1.0)[:, None]
        cat = jnp.concatenate([agg, x_ref[...]], axis=1)
        h = lax.dot_general(cat, wcat_ref[...], (((1,), (0,)), ((), ())),
                            preferred_element_type=jnp.float32)
        h = jnp.maximum(h + bl_ref[...], 0.0)

        b = batch_ref[0, 0]
        gids = lax.broadcasted_iota(jnp.int32, (1, n_graphs), 1)
        onehot = (b[:, None] == gids).astype(jnp.float32)
        sum_s[...] += lax.dot_general(onehot, h, (((0,), (0,)), ((), ())),
                                      preferred_element_type=jnp.float32)
        cnt = jnp.sum(onehot, axis=0)
        cnt_s[...] += cnt[:, None]

        gg = 8
        for t in range(n_graphs // gg):
            segs = t * gg + lax.broadcasted_iota(jnp.int32, (gg, 1), 0)
            m = b[None, :] == segs
            hb = jnp.where(m[:, :, None], h[None, :, :], -jnp.inf)
            mx = jnp.max(hb, axis=1)
            max_s[t * gg:(t + 1) * gg, :] = jnp.maximum(
                max_s[t * gg:(t + 1) * gg, :], mx)

        @pl.when(i == nsteps - 1)
        def _final():
            xm = max_s[...]
            xm = jnp.where(jnp.isfinite(xm), xm, 0.0)
            mean = sum_s[...] / jnp.maximum(cnt_s[...], 1.0)
            pooled = jnp.concatenate([xm, mean], axis=1)
            out_ref[...] = lax.dot_general(
                pooled, wlin_ref[...], (((1,), (0,)), ((), ())),
                preferred_element_type=jnp.float32) + blin_ref[...]

    return pl.pallas_call(
        body,
        grid=(nsteps,),
        in_specs=[
            pl.BlockSpec((NC, blk, d_feat), lambda i: (0, i, 0)),
            pl.BlockSpec((NC, blk, DEGW), lambda i: (0, i, 0)),
            pl.BlockSpec((blk, d_feat), lambda i: (i, 0)),
            pl.BlockSpec((1, 1, blk), lambda i: (i, 0, 0)),
            pl.BlockSpec((two_h, h_feat), lambda i: (0, 0)),
            pl.BlockSpec((1, h_feat), lambda i: (0, 0)),
            pl.BlockSpec((two_h, f_out), lambda i: (0, 0)),
            pl.BlockSpec((1, f_out), lambda i: (0, 0)),
        ],
        out_specs=pl.BlockSpec((n_graphs, f_out), lambda i: (0, 0)),
        out_shape=jax.ShapeDtypeStruct((n_graphs, f_out), jnp.float32),
        scratch_shapes=[
            pltpu.VMEM((n_graphs, h_feat), jnp.float32),
            pltpu.VMEM((n_graphs, h_feat), jnp.float32),
            pltpu.VMEM((n_graphs, h_feat), jnp.float32),
        ],
        compiler_params=pltpu.CompilerParams(
            dimension_semantics=("arbitrary",)),
    )(acc2, deg2, x, batch3, wcat, bl2, wlin_t, blin2)


def kernel(x, edge_index, batch, W_l, b_l, W_r, W_lin, b_lin):
    n_nodes, d_feat = x.shape
    n_edges = edge_index.shape[1]
    h_feat = W_l.shape[0]
    f_out = W_lin.shape[0]
    n_graphs = 64
    n_chunks = n_edges // (NW * CHUNK)

    src3 = edge_index[0].reshape(NW, n_chunks, CHUNK)
    dst3 = edge_index[1].reshape(NW, n_chunks, CHUNK)

    acc2, deg2 = _sc_aggregate(x, src3, dst3, n_nodes, d_feat, n_chunks)

    blk = 1000
    batch3 = batch.astype(jnp.int32).reshape(n_nodes // blk, 1, blk)
    wcat = jnp.concatenate([W_l, W_r], axis=1).T
    wlin_t = W_lin.T
    bl2 = b_l.reshape(1, h_feat)
    blin2 = b_lin.reshape(1, f_out)

    return _tc_fused(acc2, deg2, x, batch3, wcat, bl2, wlin_t, blin2,
                     n_nodes, d_feat, h_feat, f_out, n_graphs, blk)


# speedup vs baseline: 7.1871x; 7.1871x over previous
import functools

import jax
import jax.numpy as jnp
from jax import lax
from jax.experimental import pallas as pl
from jax.experimental.pallas import tpu as pltpu
from jax.experimental.pallas import tpu_sc as plsc

NC, NS, L = 2, 16, 16
NW = NC * NS
CHUNK = 125
DEGW = 16


def _sc_aggregate(x, src3, dst3, n_nodes, d_feat, n_chunks):
    rpt = (n_nodes // NS) & ~7
    tail = n_nodes - NS * rpt
    assert tail % 8 == 0 and tail <= rpt
    ib = 8
    assert n_chunks % ib == 0
    mesh = plsc.VectorSubcoreMesh(core_axis_name="c", subcore_axis_name="s")

    zacc = jnp.zeros((rpt, d_feat), jnp.float32)
    zdeg = jnp.zeros((rpt, DEGW), jnp.float32)

    @functools.partial(
        pl.kernel,
        mesh=mesh,
        out_type=[
            pltpu.HBM((NC, n_nodes, d_feat), jnp.float32),
            pltpu.HBM((NC, n_nodes, DEGW), jnp.float32),
        ],
        scratch_types=[
            pltpu.VMEM((ib, CHUNK), jnp.int32),
            pltpu.VMEM((ib, CHUNK), jnp.int32),
            pltpu.VMEM((CHUNK, d_feat), jnp.float32),
            pltpu.VMEM((CHUNK, DEGW), jnp.float32),
            pltpu.VMEM_SHARED((n_nodes, d_feat), jnp.float32),
            pltpu.VMEM_SHARED((n_nodes, DEGW), jnp.float32),
        ],
        compiler_params=pltpu.CompilerParams(use_tc_tiling_on_sc=False),
    )
    def agg_kernel(x_hbm, src_hbm, dst_hbm, zacc_hbm, zdeg_hbm,
                   acc_out, deg_out,
                   src_v, dst_v, rows_v, ones_v, acc_s, deg_s):
        cid = lax.axis_index("c")
        sid = lax.axis_index("s")
        wid = sid * NC + cid
        base = sid * rpt

        one16 = jnp.ones((L,), jnp.float32)

        def fill_ones(r, _):
            ones_v[r, :] = one16
            return 0
        lax.fori_loop(0, CHUNK, fill_ones, 0)

        pltpu.sync_copy(zacc_hbm, acc_s.at[pl.ds(base, rpt)])
        pltpu.sync_copy(zdeg_hbm, deg_s.at[pl.ds(base, rpt)])

        @pl.when(sid == NS - 1)
        def _zero_tail():
            pltpu.sync_copy(zacc_hbm.at[pl.ds(0, tail)],
                            acc_s.at[pl.ds(NS * rpt, tail)])
            pltpu.sync_copy(zdeg_hbm.at[pl.ds(0, tail)],
                            deg_s.at[pl.ds(NS * rpt, tail)])
        plsc.subcore_barrier()

        def block_body(b, _):
            pltpu.sync_copy(src_hbm.at[wid, pl.ds(b * ib, ib)], src_v)
            pltpu.sync_copy(dst_hbm.at[wid, pl.ds(b * ib, ib)], dst_v)

            def chunk_body(o, _):
                pltpu.sync_copy(x_hbm.at[src_v.at[o]], rows_v)
                pltpu.sync_copy(rows_v, acc_s.at[dst_v.at[o]], add=True)
                pltpu.sync_copy(ones_v, deg_s.at[dst_v.at[o]], add=True)
                return 0
            lax.fori_loop(0, ib, chunk_body, 0)
            return 0
        lax.fori_loop(0, n_chunks // ib, block_body, 0)
        plsc.subcore_barrier()

        pltpu.sync_copy(acc_s.at[pl.ds(base, rpt)],
                        acc_out.at[cid, pl.ds(base, rpt)])
        pltpu.sync_copy(deg_s.at[pl.ds(base, rpt)],
                        deg_out.at[cid, pl.ds(base, rpt)])

        @pl.when(sid == NS - 1)
        def _read_tail():
            pltpu.sync_copy(acc_s.at[pl.ds(NS * rpt, tail)],
                            acc_out.at[cid, pl.ds(NS * rpt, tail)])
            pltpu.sync_copy(deg_s.at[pl.ds(NS * rpt, tail)],
                            deg_out.at[cid, pl.ds(NS * rpt, tail)])

    return agg_kernel(x, src3, dst3, zacc, zdeg)


def _tc_fused(acc2, deg2, x, batch3, wcat, bl2, wlin_t, blin2,
              n_nodes, d_feat, h_feat, f_out, n_graphs, blk):
    nsteps = n_nodes // blk
    two_h = 2 * h_feat

    def body(acc_ref, deg_ref, x_ref, batch_ref, wcat_ref, bl_ref,
             wlin_ref, blin_ref, out_ref, max_s, sum_s, cnt_s):
        i = pl.program_id(0)

        @pl.when(i == 0)
        def _init():
            max_s[...] = jnp.full((n_graphs, h_feat), -jnp.inf, jnp.float32)
            sum_s[...] = jnp.zeros((n_graphs, h_feat), jnp.float32)
            cnt_s[...] = jnp.zeros((n_graphs, 1), jnp.float32)

        a = acc_ref[0] + acc_ref[1]
        d = deg_ref[0, :, 0:1] + deg_ref[1, :, 0:1]
        agg = a / jnp.maximum(d, 1.0)
        cat = jnp.concatenate([agg, x_ref[...]], axis=1)
        h = lax.dot_general(cat, wcat_ref[...], (((1,), (0,)), ((), ())),
                            preferred_element_type=jnp.float32)
        h = jnp.maximum(h + bl_ref[...], 0.0)

        b2 = batch_ref[0]
        gids = lax.broadcasted_iota(jnp.int32, (1, n_graphs), 1)
        onehot = (b2 == gids).astype(jnp.float32)
        sum_s[...] += lax.dot_general(onehot, h, (((0,), (0,)), ((), ())),
                                      preferred_element_type=jnp.float32)
        ones_col = jnp.ones((blk, 1), jnp.float32)
        cnt_s[...] += lax.dot_general(onehot, ones_col,
                                      (((0,), (0,)), ((), ())),
                                      preferred_element_type=jnp.float32)

        for g in range(n_graphs):
            mg = b2 == g
            hb = jnp.where(mg, h, -jnp.inf)
            mx = jnp.max(hb, axis=0, keepdims=True)
            max_s[g:g + 1, :] = jnp.maximum(max_s[g:g + 1, :], mx)

        @pl.when(i == nsteps - 1)
        def _final():
            xm = max_s[...]
            xm = jnp.where(jnp.isfinite(xm), xm, 0.0)
            mean = sum_s[...] / jnp.maximum(cnt_s[...], 1.0)
            pooled = jnp.concatenate([xm, mean], axis=1)
            out_ref[...] = lax.dot_general(
                pooled, wlin_ref[...], (((1,), (0,)), ((), ())),
                preferred_element_type=jnp.float32) + blin_ref[...]

    return pl.pallas_call(
        body,
        grid=(nsteps,),
        in_specs=[
            pl.BlockSpec((NC, blk, d_feat), lambda i: (0, i, 0)),
            pl.BlockSpec((NC, blk, DEGW), lambda i: (0, i, 0)),
            pl.BlockSpec((blk, d_feat), lambda i: (i, 0)),
            pl.BlockSpec((1, blk, 1), lambda i: (i, 0, 0)),
            pl.BlockSpec((two_h, h_feat), lambda i: (0, 0)),
            pl.BlockSpec((1, h_feat), lambda i: (0, 0)),
            pl.BlockSpec((two_h, f_out), lambda i: (0, 0)),
            pl.BlockSpec((1, f_out), lambda i: (0, 0)),
        ],
        out_specs=pl.BlockSpec((n_graphs, f_out), lambda i: (0, 0)),
        out_shape=jax.ShapeDtypeStruct((n_graphs, f_out), jnp.float32),
        scratch_shapes=[
            pltpu.VMEM((n_graphs, h_feat), jnp.float32),
            pltpu.VMEM((n_graphs, h_feat), jnp.float32),
            pltpu.VMEM((n_graphs, 1), jnp.float32),
        ],
        compiler_params=pltpu.CompilerParams(
            dimension_semantics=("arbitrary",)),
    )(acc2, deg2, x, batch3, wcat, bl2, wlin_t, blin2)


def kernel(x, edge_index, batch, W_l, b_l, W_r, W_lin, b_lin):
    n_nodes, d_feat = x.shape
    n_edges = edge_index.shape[1]
    h_feat = W_l.shape[0]
    f_out = W_lin.shape[0]
    n_graphs = 64
    n_chunks = n_edges // (NW * CHUNK)

    src3 = edge_index[0].reshape(NW, n_chunks, CHUNK)
    dst3 = edge_index[1].reshape(NW, n_chunks, CHUNK)

    acc2, deg2 = _sc_aggregate(x, src3, dst3, n_nodes, d_feat, n_chunks)

    blk = 1000
    batch3 = batch.astype(jnp.int32).reshape(n_nodes // blk, blk, 1)
    wcat = jnp.concatenate([W_l, W_r], axis=1).T
    wlin_t = W_lin.T
    bl2 = b_l.reshape(1, h_feat)
    blin2 = b_lin.reshape(1, f_out)

    return _tc_fused(acc2, deg2, x, batch3, wcat, bl2, wlin_t, blin2,
                     n_nodes, d_feat, h_feat, f_out, n_graphs, blk)
